# Initial kernel scaffold; baseline (speedup 1.0000x reference)
import jax
import jax.numpy as jnp
from jax.experimental import pallas as pl


def kernel(nbr_ids, seg_ids, seg_to_subj, s, r, ent_embeds, rel_embeds, W, b):
    raise NotImplementedError("write your pallas kernel here")


# speedup vs baseline: 1.3886x; 1.3886x over previous
import functools

import jax
import jax.numpy as jnp
from jax import lax
from jax.experimental import pallas as pl
from jax.experimental.pallas import tpu as pltpu
from jax.experimental.pallas import tpu_sc as plsc

H = 256
SEQ = 10
S = 16384
BSUBJ = 2048
N = 262144
NC = 2
NS = 16
NW = NC * NS
L = 16
KBLK = 128
CHUNKS = 64
SEGC = S // CHUNKS
ROWS_PER_W = S // NW
ACC_ROWS = SEGC + 8
FEAT_ROWS = S + KBLK
OUT_ROWS = BSUBJ * SEQ
OUT_PER_W = OUT_ROWS // NW

_mesh = plsc.VectorSubcoreMesh(core_axis_name="c", subcore_axis_name="s",
                               num_cores=NC, num_subcores=NS)


def _zero_vmem(ref, nrows, ncols):
    z = jnp.zeros((L,), jnp.float32)

    def body(i, carry):
        for j in range(ncols // L):
            ref[i, pl.ds(j * L, L)] = z
        return carry

    lax.fori_loop(0, nrows, body, 0)


@functools.partial(
    pl.kernel,
    out_type=(
        jax.ShapeDtypeStruct((S, H), jnp.float32),
        jax.ShapeDtypeStruct((S, H), jnp.float32),
    ),
    mesh=_mesh,
    scratch_types=[
        pltpu.VMEM((KBLK,), jnp.int32),
        pltpu.VMEM((KBLK,), jnp.int32),
        pltpu.VMEM((KBLK,), jnp.int32),
        pltpu.VMEM((KBLK, H), jnp.float32),
        pltpu.VMEM((ACC_ROWS, H), jnp.float32),
        pltpu.VMEM((ACC_ROWS + L,), jnp.float32),
        pltpu.VMEM((80,), jnp.int32),
        pltpu.SemaphoreType.DMA,
    ],
)
def _sc_aggregate(nbr_hbm, seg_hbm, starts_hbm, subj_hbm, ent_hbm,
                  means_hbm, subjfeat_hbm,
                  idx_v, seg_v, cl_v, rows_v, acc_v, cnt_acc, starts_v, sem):
    cid = lax.axis_index("c")
    sid = lax.axis_index("s")
    wid = sid * NC + cid

    pltpu.sync_copy(starts_hbm, starts_v)

    def sload(idx):
        return starts_v[pl.ds(idx, L)][0]

    lane0 = jnp.where(lax.iota(jnp.int32, L) == 0, 1.0, 0.0)

    for j in range(CHUNKS // NW):
        chunk = wid * (CHUNKS // NW) + j
        seg_base = chunk * SEGC
        st = sload(chunk)
        en = sload(chunk + 1)
        b0 = st // KBLK
        b1 = (en + KBLK - 1) // KBLK

        _zero_vmem(acc_v, ACC_ROWS, H)
        z = jnp.zeros((L,), jnp.float32)

        def zc_body(i, c):
            cnt_acc[pl.ds(i * L, L)] = z
            return c

        lax.fori_loop(0, (ACC_ROWS + L) // L, zc_body, 0)

        def blk_body(b, carry):
            off = b * KBLK
            pltpu.sync_copy(seg_hbm.at[pl.ds(off, KBLK)], seg_v)
            pltpu.sync_copy(nbr_hbm.at[pl.ds(off, KBLK)], idx_v)
            for q in range(KBLK // L):
                v = seg_v[pl.ds(q * L, L)]
                local = v - seg_base
                ok = (local >= 0) & (local < SEGC)
                cl_v[pl.ds(q * L, L)] = jnp.where(ok, local, SEGC)
            pltpu.async_copy(ent_hbm.at[idx_v], rows_v, sem).wait()

            def row_body(i, c):
                ci = cl_v[pl.ds(i, L)][0]
                for q in range(H // L):
                    acc_v[ci, pl.ds(q * L, L)] = (
                        acc_v[ci, pl.ds(q * L, L)] + rows_v[i, pl.ds(q * L, L)])
                cnt_acc[pl.ds(ci, L)] = cnt_acc[pl.ds(ci, L)] + lane0
                return c

            lax.fori_loop(0, KBLK, row_body, 0)
            return carry

        lax.fori_loop(b0, b1, blk_body, 0)

        def div_body(i, c):
            cvec = jnp.full((L,), cnt_acc[pl.ds(i, L)][0], jnp.float32)
            recip = 1.0 / jnp.maximum(cvec, 1.0)
            for q in range(H // L):
                acc_v[i, pl.ds(q * L, L)] = acc_v[i, pl.ds(q * L, L)] * recip
            return c

        lax.fori_loop(0, SEGC, div_body, 0)
        pltpu.sync_copy(acc_v.at[pl.ds(0, SEGC)],
                        means_hbm.at[pl.ds(seg_base, SEGC)])

    row0 = wid * ROWS_PER_W
    for t in range(ROWS_PER_W // KBLK):
        r0 = row0 + t * KBLK
        pltpu.sync_copy(subj_hbm.at[pl.ds(r0, KBLK)], idx_v)
        pltpu.async_copy(ent_hbm.at[idx_v], rows_v, sem).wait()
        pltpu.sync_copy(rows_v, subjfeat_hbm.at[pl.ds(r0, KBLK)])


def _tc_gcn_body(means_ref, subj_ref, w_ref, b_ref, out_ref):
    i = pl.program_id(0)
    x = means_ref[...]
    g = jnp.dot(x, w_ref[...], preferred_element_type=jnp.float32)
    g = jnp.maximum(g + b_ref[...], 0.0)
    feat = jnp.concatenate([g, subj_ref[...]], axis=1)
    out_ref[...] = jnp.where(i < S // KBLK, feat, 0.0)


_NBLK = S // KBLK


def _tc_gcn(means, subjfeat, W, b2):
    clamp = lambda i: (jnp.minimum(i, _NBLK - 1), 0)
    return pl.pallas_call(
        _tc_gcn_body,
        grid=(_NBLK + 1,),
        in_specs=[
            pl.BlockSpec((KBLK, H), clamp),
            pl.BlockSpec((KBLK, H), clamp),
            pl.BlockSpec((H, H), lambda i: (0, 0)),
            pl.BlockSpec((1, H), lambda i: (0, 0)),
        ],
        out_specs=pl.BlockSpec((KBLK, 2 * H), lambda i: (i, 0)),
        out_shape=jax.ShapeDtypeStruct((FEAT_ROWS, 2 * H), jnp.float32),
    )(means, subjfeat, W, b2)


@functools.partial(
    pl.kernel,
    out_type=jax.ShapeDtypeStruct((OUT_ROWS, 2 * H), jnp.float32),
    mesh=_mesh,
    scratch_types=[
        pltpu.VMEM((KBLK,), jnp.int32),
        pltpu.VMEM((KBLK, 2 * H), jnp.float32),
        pltpu.SemaphoreType.DMA,
    ],
)
def _sc_emit(feat_hbm, oidx_hbm, out_hbm, idx_v, rows_v, sem):
    cid = lax.axis_index("c")
    sid = lax.axis_index("s")
    wid = sid * NC + cid
    base = wid * OUT_PER_W
    for t in range(OUT_PER_W // KBLK):
        r0 = base + t * KBLK
        pltpu.sync_copy(oidx_hbm.at[pl.ds(r0, KBLK)], idx_v)
        pltpu.async_copy(feat_hbm.at[idx_v], rows_v, sem).wait()
        pltpu.sync_copy(rows_v, out_hbm.at[pl.ds(r0, KBLK)])


def kernel(nbr_ids, seg_ids, seg_to_subj, s, r, ent_embeds, rel_embeds, W, b):
    nbr_ids = nbr_ids.astype(jnp.int32)
    seg_ids = seg_ids.astype(jnp.int32)
    seg_to_subj = seg_to_subj.astype(jnp.int32)
    s = s.astype(jnp.int32)

    nbr_pad = jnp.concatenate([nbr_ids, jnp.zeros((KBLK,), jnp.int32)])
    seg_pad = jnp.concatenate([seg_ids, jnp.full((KBLK,), S, jnp.int32)])
    bounds = jnp.arange(0, S + 1, SEGC, dtype=jnp.int32)
    starts = jnp.searchsorted(seg_ids, bounds).astype(jnp.int32)
    starts = jnp.concatenate([starts, jnp.zeros((80 - CHUNKS - 1,), jnp.int32)])
    subj_idx = jnp.take(s, seg_to_subj)

    rows = jnp.arange(OUT_ROWS, dtype=jnp.int32)
    subj = rows // SEQ
    p = rows % SEQ
    first = jnp.searchsorted(seg_to_subj, jnp.arange(BSUBJ + 1, dtype=jnp.int32)
                             ).astype(jnp.int32)
    cnt_subj = jnp.take(first, subj + 1) - jnp.take(first, subj)
    zero_row = S + (rows % KBLK)
    out_idx = jnp.where(p < cnt_subj, jnp.take(first, subj) + p, zero_row)

    means, subjfeat = _sc_aggregate(nbr_pad, seg_pad, starts, subj_idx,
                                    ent_embeds)
    feat = _tc_gcn(means, subjfeat, W, b.reshape(1, H))
    out = _sc_emit(feat, out_idx)
    return out.reshape(BSUBJ, SEQ, 2 * H)


# speedup vs baseline: 2.4670x; 1.7767x over previous
import functools

import jax
import jax.numpy as jnp
from jax import lax
from jax.experimental import pallas as pl
from jax.experimental.pallas import tpu as pltpu
from jax.experimental.pallas import tpu_sc as plsc

H = 256
SEQ = 10
S = 16384
BSUBJ = 2048
N = 262144
NC = 2
NS = 16
NW = NC * NS
L = 16
KBLK = 128
CHUNKS = 128
SEGC = S // CHUNKS
ROWS_PER_W = S // NW
ACC_ROWS = SEGC + 8
FEAT_ROWS = S + KBLK
OUT_ROWS = BSUBJ * SEQ
OUT_PER_W = OUT_ROWS // NW

_mesh = plsc.VectorSubcoreMesh(core_axis_name="c", subcore_axis_name="s",
                               num_cores=NC, num_subcores=NS)


def _zero_vmem(ref, nrows, ncols):
    z = jnp.zeros((L,), jnp.float32)

    def body(i, carry):
        for j in range(ncols // L):
            ref[i, pl.ds(j * L, L)] = z
        return carry

    lax.fori_loop(0, nrows, body, 0)


@functools.partial(
    pl.kernel,
    out_type=(
        jax.ShapeDtypeStruct((S, H), jnp.float32),
        jax.ShapeDtypeStruct((S, H), jnp.float32),
    ),
    mesh=_mesh,
    compiler_params=pltpu.CompilerParams(needs_layout_passes=False),
    scratch_types=[
        pltpu.VMEM((KBLK,), jnp.int32),
        pltpu.VMEM((KBLK,), jnp.int32),
        pltpu.VMEM((KBLK,), jnp.int32),
        pltpu.VMEM((KBLK,), jnp.int32),
        pltpu.VMEM((KBLK,), jnp.int32),
        pltpu.VMEM((KBLK, H), jnp.float32),
        pltpu.VMEM((KBLK, H), jnp.float32),
        pltpu.VMEM((ACC_ROWS * H,), jnp.float32),
        pltpu.VMEM((160,), jnp.int32),
        pltpu.VMEM((BSUBJ,), jnp.float32),
        pltpu.SemaphoreType.DMA,
        pltpu.SemaphoreType.DMA,
    ],
)
def _sc_aggregate(nbr_hbm, seg_hbm, starts_hbm, s2s_hbm, s_hbm, ent_hbm,
                  means_hbm, subjfeat_hbm,
                  idxb0, idxb1, segb0, segb1, cl_v, rowsb0, rowsb1,
                  acc_v, starts_v, s_v, sem0, sem1):
    cid = lax.axis_index("c")
    sid = lax.axis_index("s")
    wid = sid * NC + cid

    pltpu.sync_copy(starts_hbm, starts_v)

    def sload(idx):
        return starts_v[pl.ds(idx, L)][0]

    bufs = ((idxb0, segb0, rowsb0, sem0), (idxb1, segb1, rowsb1, sem1))

    def chunk_body(j, _):
        chunk = wid * (CHUNKS // NW) + j
        seg_base = chunk * SEGC
        st = sload(chunk)
        en = sload(chunk + 1)
        b0 = st // KBLK
        nb = (en + KBLK - 1) // KBLK - b0

        z16 = jnp.zeros((L,), jnp.float32)

        def za_body(i, c):
            acc_v[pl.ds(i * L, L)] = z16
            return c

        lax.fori_loop(0, ACC_ROWS * H // L, za_body, 0)

        @pl.when(nb > 0)
        def _():
            off = b0 * KBLK
            pltpu.sync_copy(seg_hbm.at[pl.ds(off, KBLK)], segb0)
            pltpu.sync_copy(nbr_hbm.at[pl.ds(off, KBLK)], idxb0)
            pltpu.async_copy(ent_hbm.at[idxb0], rowsb0, sem0)

        zero16 = jnp.zeros((L,), jnp.float32)
        init = (jnp.int32(SEGC), zero16) + (zero16,) * (H // L)

        def g_body(g, carry):
            for par in range(2):
                r = 2 * g + par
                idxb, segb, rowsb, semc = bufs[par]
                nidxb, nsegb, nrowsb, nsem = bufs[1 - par]

                @pl.when(r + 1 < nb)
                def _():
                    off = (b0 + r + 1) * KBLK
                    pltpu.sync_copy(seg_hbm.at[pl.ds(off, KBLK)], nsegb)
                    pltpu.sync_copy(nbr_hbm.at[pl.ds(off, KBLK)], nidxb)
                    pltpu.async_copy(ent_hbm.at[nidxb], nrowsb, nsem)

                @pl.when(r < nb)
                def _():
                    pltpu.make_async_copy(ent_hbm.at[idxb], rowsb, semc).wait()

                base2 = jnp.where(r < nb, seg_base, jnp.int32(-2147000000))
                for q in range(KBLK // L):
                    v = segb[pl.ds(q * L, L)]
                    local = v - base2
                    ok = (local >= 0) & (local < SEGC)
                    cl_v[pl.ds(q * L, L)] = jnp.where(ok, local, SEGC)

                def row_body(i, c):
                    ci, cnt = c[0], c[1]
                    acc = c[2:]
                    nci = cl_v[pl.ds(i, L)][0]
                    changed = nci != ci

                    @pl.when(changed)
                    def _():
                        recip = 1.0 / jnp.maximum(cnt, 1.0)
                        for q in range(H // L):
                            acc_v[pl.ds(ci * H + q * L, L)] = acc[q] * recip

                    keep = jnp.full((L,), jnp.where(changed, 0.0, 1.0),
                                    jnp.float32)
                    cnt2 = keep * cnt + 1.0
                    acc2 = tuple(
                        keep * acc[q] + rowsb[i, pl.ds(q * L, L)]
                        for q in range(H // L))
                    return (nci, cnt2) + acc2

                carry = lax.fori_loop(0, KBLK, row_body, carry)
            return carry

        ng = (nb + 1) // 2
        carry = lax.fori_loop(0, ng, g_body, init)

        ci, cnt = carry[0], carry[1]
        recip = 1.0 / jnp.maximum(cnt, 1.0)
        for q in range(H // L):
            acc_v[pl.ds(ci * H + q * L, L)] = carry[2 + q] * recip

        def cp_body(i, c):
            for q in range(H // L):
                rowsb0[i, pl.ds(q * L, L)] = acc_v[pl.ds(i * H + q * L, L)]
            return c

        lax.fori_loop(0, SEGC, cp_body, 0)
        pltpu.sync_copy(rowsb0, means_hbm.at[pl.ds(seg_base, SEGC)])
        return 0

    lax.fori_loop(0, CHUNKS // NW, chunk_body, 0)

    pltpu.sync_copy(s_hbm, s_v)
    row0 = wid * ROWS_PER_W
    for t in range(ROWS_PER_W // KBLK):
        r0 = row0 + t * KBLK
        pltpu.sync_copy(s2s_hbm.at[pl.ds(r0, KBLK)], idxb1)
        for q in range(KBLK // L):
            iv = idxb1[pl.ds(q * L, L)]
            sval = plsc.load_gather(s_v, [iv])
            idxb0[pl.ds(q * L, L)] = sval.astype(jnp.int32)
        pltpu.async_copy(ent_hbm.at[idxb0], rowsb0, sem0).wait()
        pltpu.sync_copy(rowsb0, subjfeat_hbm.at[pl.ds(r0, KBLK)])


def _tc_gcn_body(means_ref, subj_ref, w_ref, b_ref, out_ref):
    i = pl.program_id(0)
    x = means_ref[...]
    g = jnp.dot(x, w_ref[...], preferred_element_type=jnp.float32)
    g = jnp.maximum(g + b_ref[...], 0.0)
    feat = jnp.concatenate([g, subj_ref[...]], axis=1)
    out_ref[...] = jnp.where(i < S // KBLK, feat, 0.0)


_NBLK = S // KBLK


def _tc_gcn(means, subjfeat, W, b2):
    clamp = lambda i: (jnp.minimum(i, _NBLK - 1), 0)
    return pl.pallas_call(
        _tc_gcn_body,
        grid=(_NBLK + 1,),
        in_specs=[
            pl.BlockSpec((KBLK, H), clamp),
            pl.BlockSpec((KBLK, H), clamp),
            pl.BlockSpec((H, H), lambda i: (0, 0)),
            pl.BlockSpec((1, H), lambda i: (0, 0)),
        ],
        out_specs=pl.BlockSpec((KBLK, 2 * H), lambda i: (i, 0)),
        out_shape=jax.ShapeDtypeStruct((FEAT_ROWS, 2 * H), jnp.float32),
    )(means, subjfeat, W, b2)


@functools.partial(
    pl.kernel,
    out_type=jax.ShapeDtypeStruct((OUT_ROWS, 2 * H), jnp.float32),
    mesh=_mesh,
    compiler_params=pltpu.CompilerParams(needs_layout_passes=False),
    scratch_types=[
        pltpu.VMEM((S,), jnp.float32),
        pltpu.VMEM((80,), jnp.float32),
        pltpu.VMEM((KBLK,), jnp.int32),
        pltpu.VMEM((KBLK, 2 * H), jnp.float32),
        pltpu.SemaphoreType.DMA,
    ],
)
def _sc_emit(feat_hbm, s2s_hbm, out_hbm, s2s_v, first_tab, idx_v, rows_v, sem):
    cid = lax.axis_index("c")
    sid = lax.axis_index("s")
    wid = sid * NC + cid

    pltpu.sync_copy(s2s_hbm, s2s_v)
    subj0 = wid * (BSUBJ // NW)
    iota = lax.iota(jnp.int32, L)

    for g in range(5):
        target = (subj0 + g * L + iota).astype(jnp.float32)
        lo = jnp.zeros((L,), jnp.int32)
        hi = jnp.full((L,), S, jnp.int32)
        for _ in range(14):
            mid = (lo + hi) >> 1
            vals = plsc.load_gather(s2s_v, [mid])
            less = vals < target
            lo = jnp.where(less, mid + 1, lo)
            hi = jnp.where(less, hi, mid)
        first_tab[pl.ds(g * L, L)] = lo.astype(jnp.float32)

    base = wid * OUT_PER_W
    for t in range(OUT_PER_W // KBLK):
        r0 = base + t * KBLK
        for q in range(KBLK // L):
            r = r0 + q * L + iota
            subj = (r * 52429) >> 19
            p = r - subj * 10
            loc = subj - subj0
            f = plsc.load_gather(first_tab, [loc])
            f1 = plsc.load_gather(first_tab, [loc + 1])
            pf = p.astype(jnp.float32)
            zr = (S + (r & (KBLK - 1))).astype(jnp.float32)
            sel = jnp.where(pf < f1 - f, f + pf, zr)
            idx_v[pl.ds(q * L, L)] = sel.astype(jnp.int32)
        pltpu.async_copy(feat_hbm.at[idx_v], rows_v, sem).wait()
        pltpu.sync_copy(rows_v, out_hbm.at[pl.ds(r0, KBLK)])


def kernel(nbr_ids, seg_ids, seg_to_subj, s, r, ent_embeds, rel_embeds, W, b):
    nbr_ids = nbr_ids.astype(jnp.int32)
    seg_ids = seg_ids.astype(jnp.int32)
    seg_to_subj = seg_to_subj.astype(jnp.int32)
    s = s.astype(jnp.int32)

    bounds = jnp.arange(0, S + 1, SEGC, dtype=jnp.int32)
    starts = jnp.searchsorted(seg_ids, bounds).astype(jnp.int32)
    starts = jnp.concatenate([starts, jnp.zeros((160 - CHUNKS - 1,), jnp.int32)])

    means, subjfeat = _sc_aggregate(nbr_ids, seg_ids, starts, seg_to_subj,
                                    s.astype(jnp.float32), ent_embeds)
    feat = _tc_gcn(means, subjfeat, W, b.reshape(1, H))
    out = _sc_emit(feat, seg_to_subj.astype(jnp.float32))
    return out.reshape(BSUBJ, SEQ, 2 * H)
